# baseline (speedup 1.0000x reference)
import functools

import jax
import jax.numpy as jnp
from jax import lax
from jax.experimental import pallas as pl
from jax.experimental.pallas import tpu as pltpu
from jax.experimental.pallas import tpu_sc as plsc

N_NODES = 10000
N_EDGES = 320000
D = 128

NC = 2
NS = 16
CHUNK = 128
NBLK = 4
BCHUNK = 40
EDGES_PER_T = NBLK * BCHUNK * CHUNK
E_PAD = NS * EDGES_PER_T
ROWS_SC = 5120
ACC_ROWS = ROWS_SC + 8
ROWS_PER_TILE = ROWS_SC // NS
ZCHUNK = 64


def _sc_aggregate(feature, src, dst, vals):
    mesh = plsc.VectorSubcoreMesh(core_axis_name="c", subcore_axis_name="s")

    @functools.partial(
        pl.kernel,
        out_type=jax.ShapeDtypeStruct((NC, ROWS_SC, D), jnp.float32),
        mesh=mesh,
        scratch_types=[
            pltpu.VMEM((2, BCHUNK, CHUNK), jnp.int32),
            pltpu.VMEM((2, BCHUNK, CHUNK), jnp.int32),
            pltpu.VMEM((2, BCHUNK, CHUNK), jnp.float32),
            pltpu.VMEM((2, CHUNK, D), jnp.float32),
            pltpu.VMEM((ZCHUNK, D), jnp.float32),
            pltpu.VMEM_SHARED((ACC_ROWS, D), jnp.float32),
            pltpu.SemaphoreType.DMA,
            pltpu.SemaphoreType.DMA,
            pltpu.SemaphoreType.DMA,
            pltpu.SemaphoreType.DMA,
            pltpu.SemaphoreType.DMA,
        ],
    )
    def k(feat_hbm, src_hbm, dst_hbm, vals_hbm, out_hbm,
          src_v, dst_v, vals_v, rows_v, zero_v, acc_sh,
          esem, g0, g1, t0, t1):
        cid = lax.axis_index("c")
        sid = lax.axis_index("s")
        gsem = (g0, g1)
        tsem = (t0, t1)

        zero16 = jnp.zeros((16,), jnp.float32)

        def zbody(i, carry):
            for j in range(D // 16):
                zero_v[i, pl.ds(j * 16, 16)] = zero16
            return carry

        lax.fori_loop(0, ZCHUNK, zbody, 0)
        for z in range(ROWS_PER_TILE // ZCHUNK):
            pltpu.sync_copy(
                zero_v,
                acc_sh.at[pl.ds(sid * ROWS_PER_TILE + z * ZCHUNK, ZCHUNK)])

        @pl.when(sid == 0)
        def _zero_dump():
            pltpu.sync_copy(zero_v.at[pl.ds(0, 8)],
                            acc_sh.at[pl.ds(ROWS_SC, 8)])

        plsc.subcore_barrier()

        base = cid * ROWS_SC

        def stage(blk, eb):
            pltpu.async_copy(src_hbm.at[sid].at[blk], src_v.at[eb], esem)
            pltpu.async_copy(dst_hbm.at[sid].at[blk], dst_v.at[eb], esem)
            pltpu.async_copy(vals_hbm.at[sid].at[blk], vals_v.at[eb], esem)

        def stage_wait(blk, eb):
            pltpu.make_async_copy(
                src_hbm.at[sid].at[blk], src_v.at[eb], esem).wait()
            pltpu.make_async_copy(
                dst_hbm.at[sid].at[blk], dst_v.at[eb], esem).wait()
            pltpu.make_async_copy(
                vals_hbm.at[sid].at[blk], vals_v.at[eb], esem).wait()

        def gather_start(eb, c, b):
            pltpu.async_copy(
                feat_hbm.at[src_v.at[eb].at[c]], rows_v.at[b], gsem[b])

        def gather_wait(eb, c, b):
            pltpu.make_async_copy(
                feat_hbm.at[src_v.at[eb].at[c]], rows_v.at[b],
                gsem[b]).wait()

        def scatter_start(eb, c, b):
            pltpu.async_copy(
                rows_v.at[b], acc_sh.at[dst_v.at[eb].at[c]], tsem[b],
                add=True)

        def scatter_wait(eb, c, b):
            pltpu.make_async_copy(
                rows_v.at[b], acc_sh.at[dst_v.at[eb].at[c]],
                tsem[b]).wait()

        def scale(eb, c, b):
            def scale_group(g, carry):
                bvals = vals_v[eb, c, pl.ds(g * 16, 16)]
                for l in range(16):
                    bval = jnp.broadcast_to(bvals[l], (16,))
                    for j in range(D // 16):
                        sl = pl.ds(j * 16, 16)
                        rows_v[b, g * 16 + l, sl] = (
                            rows_v[b, g * 16 + l, sl] * bval)
                return carry

            lax.fori_loop(0, CHUNK // 16, scale_group, 0)

        stage(0, 0)

        def blk_body(blk, carry):
            eb = lax.rem(blk, 2)
            stage_wait(blk, eb)

            @pl.when(blk + 1 < NBLK)
            def _stage_next():
                stage(blk + 1, 1 - eb)

            def rw_body(r, carry2):
                for j in range(CHUNK // 16):
                    sl = pl.ds(j * 16, 16)
                    d2 = dst_v[eb, r, sl] - base
                    ok = (d2 >= 0) & (d2 < ROWS_SC)
                    dst_v[eb, r, sl] = jnp.where(ok, d2, ROWS_SC)
                return carry2

            lax.fori_loop(0, BCHUNK, rw_body, 0)

            gather_start(eb, 0, 0)

            def pair_body(p, carry2):
                c0 = 2 * p
                c1 = c0 + 1
                gather_start(eb, c1, 1)
                gather_wait(eb, c0, 0)
                scale(eb, c0, 0)
                scatter_start(eb, c0, 0)
                gather_wait(eb, c1, 1)
                scale(eb, c1, 1)
                scatter_start(eb, c1, 1)
                scatter_wait(eb, c0, 0)

                @pl.when(p + 1 < BCHUNK // 2)
                def _next_gather():
                    gather_start(eb, c0 + 2, 0)

                scatter_wait(eb, c1, 1)
                return carry2

            lax.fori_loop(0, BCHUNK // 2, pair_body, 0)
            return carry

        lax.fori_loop(0, NBLK, blk_body, 0)
        plsc.subcore_barrier()

        rsl = pl.ds(sid * ROWS_PER_TILE, ROWS_PER_TILE)
        pltpu.sync_copy(acc_sh.at[rsl], out_hbm.at[cid].at[rsl])

    return k(feature, src, dst, vals)


def _tc_combine(partials, weight, bias2d):
    BR = 512
    NB = ROWS_SC // BR

    def body(p_ref, w_ref, b_ref, o_ref):
        y = jnp.dot(p_ref[0], w_ref[...], preferred_element_type=jnp.float32)
        o_ref[...] = jnp.maximum(y + b_ref[...], 0.0)

    return pl.pallas_call(
        body,
        grid=(NC * NB,),
        in_specs=[
            pl.BlockSpec((1, BR, D), lambda i: (i // NB, i % NB, 0)),
            pl.BlockSpec((D, D), lambda i: (0, 0)),
            pl.BlockSpec((1, D), lambda i: (0, 0)),
        ],
        out_specs=pl.BlockSpec((BR, D), lambda i: (i, 0)),
        out_shape=jax.ShapeDtypeStruct((N_NODES, D), jnp.float32),
    )(partials, weight, bias2d)


def kernel(feature, edge_index, edge_values, weight, bias):
    eshape = (NS, NBLK, BCHUNK, CHUNK)
    npad = E_PAD - N_EDGES
    src = jnp.concatenate(
        [edge_index[0].astype(jnp.int32),
         jnp.zeros((npad,), jnp.int32)]).reshape(eshape)
    dst = jnp.concatenate(
        [edge_index[1].astype(jnp.int32),
         jnp.full((npad,), -1, jnp.int32)]).reshape(eshape)
    vals = jnp.concatenate(
        [edge_values, jnp.zeros((npad,), jnp.float32)]).reshape(eshape)
    partials = _sc_aggregate(feature, src, dst, vals)
    return _tc_combine(partials, weight, bias.reshape(1, D))


# speedup vs baseline: 2.5243x; 2.5243x over previous
import functools

import jax
import jax.numpy as jnp
from jax import lax
from jax.experimental import pallas as pl
from jax.experimental.pallas import tpu as pltpu
from jax.experimental.pallas import tpu_sc as plsc

N_NODES = 10000
N_EDGES = 320000
D = 128

NC = 2
NS = 16
EDGES_PER_T = N_EDGES // NS
CHUNK = 80
NBLK = 5
BCHUNK = EDGES_PER_T // (NBLK * CHUNK)
ROWS_SC = 5120
ACC_ROWS = ROWS_SC + 8
ROWS_PER_TILE = ROWS_SC // NS
ZCHUNK = 64


def _sc_aggregate(feature, src, dst, vals):
    mesh = plsc.VectorSubcoreMesh(core_axis_name="c", subcore_axis_name="s")

    @functools.partial(
        pl.kernel,
        out_type=jax.ShapeDtypeStruct((NC, ROWS_SC, D), jnp.float32),
        mesh=mesh,
        scratch_types=[
            pltpu.VMEM((BCHUNK, CHUNK), jnp.int32),
            pltpu.VMEM((BCHUNK, CHUNK), jnp.int32),
            pltpu.VMEM((BCHUNK, CHUNK), jnp.float32),
            pltpu.VMEM((2, CHUNK, D), jnp.float32),
            pltpu.VMEM((ZCHUNK, D), jnp.float32),
            pltpu.VMEM_SHARED((ACC_ROWS, D), jnp.float32),
            pltpu.SemaphoreType.DMA,
            pltpu.SemaphoreType.DMA,
        ],
    )
    def k(feat_hbm, src_hbm, dst_hbm, vals_hbm, out_hbm,
          src_v, dst_v, vals_v, rows_v, zero_v, acc_sh, g0, g1):
        cid = lax.axis_index("c")
        sid = lax.axis_index("s")
        gsem = (g0, g1)

        zero16 = jnp.zeros((16,), jnp.float32)

        def zbody(i, carry):
            for j in range(D // 16):
                zero_v[i, pl.ds(j * 16, 16)] = zero16
            return carry

        lax.fori_loop(0, ZCHUNK, zbody, 0)
        for z in range(ROWS_PER_TILE // ZCHUNK):
            pltpu.sync_copy(
                zero_v,
                acc_sh.at[pl.ds(sid * ROWS_PER_TILE + z * ZCHUNK, ZCHUNK)])

        @pl.when(sid == 0)
        def _zero_dump():
            pltpu.sync_copy(zero_v.at[pl.ds(0, 8)],
                            acc_sh.at[pl.ds(ROWS_SC, 8)])

        plsc.subcore_barrier()

        base = cid * ROWS_SC

        def gather_start(c, b):
            pltpu.async_copy(
                feat_hbm.at[src_v.at[c]], rows_v.at[b], gsem[b])

        def gather_wait(c, b):
            pltpu.make_async_copy(
                feat_hbm.at[src_v.at[c]], rows_v.at[b], gsem[b]).wait()

        def scale(c, b):
            def scale_group(g, carry):
                bvals = vals_v[c, pl.ds(g * 16, 16)]
                for l in range(16):
                    bval = jnp.broadcast_to(bvals[l], (16,))
                    for j in range(D // 16):
                        sl = pl.ds(j * 16, 16)
                        rows_v[b, g * 16 + l, sl] = (
                            rows_v[b, g * 16 + l, sl] * bval)
                return carry

            lax.fori_loop(0, CHUNK // 16, scale_group, 0)

        def scatter(c, b):
            pltpu.sync_copy(rows_v.at[b], acc_sh.at[dst_v.at[c]], add=True)

        def blk_body(blk, carry):
            pltpu.sync_copy(src_hbm.at[sid].at[blk], src_v)
            pltpu.sync_copy(dst_hbm.at[sid].at[blk], dst_v)
            pltpu.sync_copy(vals_hbm.at[sid].at[blk], vals_v)

            def rw_body(r, carry2):
                for j in range(CHUNK // 16):
                    sl = pl.ds(j * 16, 16)
                    d2 = dst_v[r, sl] - base
                    ok = (d2 >= 0) & (d2 < ROWS_SC)
                    dst_v[r, sl] = jnp.where(ok, d2, ROWS_SC)
                return carry2

            lax.fori_loop(0, BCHUNK, rw_body, 0)

            gather_start(0, 0)

            def pair_body(p, carry2):
                c0 = 2 * p
                c1 = c0 + 1
                gather_start(c1, 1)
                gather_wait(c0, 0)
                scale(c0, 0)
                scatter(c0, 0)

                @pl.when(p + 1 < BCHUNK // 2)
                def _next_gather():
                    gather_start(c0 + 2, 0)

                gather_wait(c1, 1)
                scale(c1, 1)
                scatter(c1, 1)
                return carry2

            lax.fori_loop(0, BCHUNK // 2, pair_body, 0)
            return carry

        lax.fori_loop(0, NBLK, blk_body, 0)
        plsc.subcore_barrier()

        rsl = pl.ds(sid * ROWS_PER_TILE, ROWS_PER_TILE)
        pltpu.sync_copy(acc_sh.at[rsl], out_hbm.at[cid].at[rsl])

    return k(feature, src, dst, vals)


def _tc_combine(partials, weight, bias2d):
    BR = 512
    NB = ROWS_SC // BR

    def body(p_ref, w_ref, b_ref, o_ref):
        y = jnp.dot(p_ref[0], w_ref[...], preferred_element_type=jnp.float32)
        o_ref[...] = jnp.maximum(y + b_ref[...], 0.0)

    return pl.pallas_call(
        body,
        grid=(NC * NB,),
        in_specs=[
            pl.BlockSpec((1, BR, D), lambda i: (i // NB, i % NB, 0)),
            pl.BlockSpec((D, D), lambda i: (0, 0)),
            pl.BlockSpec((1, D), lambda i: (0, 0)),
        ],
        out_specs=pl.BlockSpec((BR, D), lambda i: (i, 0)),
        out_shape=jax.ShapeDtypeStruct((N_NODES, D), jnp.float32),
    )(partials, weight, bias2d)


def kernel(feature, edge_index, edge_values, weight, bias):
    eshape = (NS, NBLK, BCHUNK, CHUNK)
    src = edge_index[0].astype(jnp.int32).reshape(eshape)
    dst = edge_index[1].astype(jnp.int32).reshape(eshape)
    vals = edge_values.reshape(eshape)
    partials = _sc_aggregate(feature, src, dst, vals)
    return _tc_combine(partials, weight, bias.reshape(1, D))
